# baseline (speedup 1.0000x reference)
import functools

import jax
import jax.numpy as jnp
from jax import lax
from jax.experimental import pallas as pl
from jax.experimental.pallas import tpu as pltpu
from jax.experimental.pallas import tpu_sc as plsc

NUM_EMB = 1024
DIM = 64
N_TOK = 4 * 8 * 32 * 32
TILE = 512
N_TILES = N_TOK // TILE
NW = 32
TOK_PER_W = N_TOK // NW
HALF = TOK_PER_W // 2


def _argmin_body(x_ref, w_ref, idx_ref, loss_ref):
    x = x_ref[0]
    xt = x.T
    w = w_ref[...]
    mm = lax.dot_general(
        xt, w, (((1,), (1,)), ((), ())),
        preferred_element_type=jnp.float32,
        precision=lax.Precision.DEFAULT,
    )
    xsq = jnp.sum(xt * xt, axis=1, keepdims=True)
    wsq = jnp.sum(w * w, axis=1)[None, :]
    d = (xsq + wsq) - 2.0 * mm
    mind = jnp.min(d, axis=1, keepdims=True)
    iota = lax.broadcasted_iota(jnp.int32, d.shape, 1)
    pick = jnp.where(d == mind, iota, jnp.int32(2**30))
    idx_ref[0, 0, :] = jnp.min(pick, axis=1)

    @pl.when(pl.program_id(0) == 0)
    def _():
        loss_ref[...] = jnp.zeros((1, 1), jnp.float32)

    loss_ref[...] += jnp.sum(mind).reshape(1, 1)


def _sc_body(idx_hbm, w_hbm, outq_hbm, hist_hbm, idx_v, w_v, buf_v, hist_v):
    cid = lax.axis_index("c")
    sid = lax.axis_index("s")
    wid = sid * 2 + cid
    pltpu.sync_copy(w_hbm, w_v)
    base = wid * TOK_PER_W
    pltpu.sync_copy(idx_hbm.at[pl.ds(base, TOK_PER_W)], idx_v)

    def _zero(i, carry):
        hist_v[pl.ds(i * 16, 16)] = jnp.zeros((16,), jnp.float32)
        return carry

    lax.fori_loop(0, NUM_EMB // 16, _zero, 0)

    ones = jnp.ones((16,), jnp.float32)

    def _hist(i, carry):
        iv = idx_v[pl.ds(i * 16, 16)]
        plsc.addupdate_scatter(hist_v, [iv], ones)
        return carry

    lax.fori_loop(0, TOK_PER_W // 16, _hist, 0)
    pltpu.sync_copy(hist_v, hist_hbm.at[wid])

    b = wid // 8
    off = (wid % 8) * TOK_PER_W
    for h in range(2):
        def _gather(j, carry):
            iv = idx_v[pl.ds(h * HALF + j * 16, 16)] * DIM
            for c in range(DIM):
                buf_v[c, pl.ds(j * 16, 16)] = plsc.load_gather(w_v, [iv + c])
            return carry

        lax.fori_loop(0, HALF // 16, _gather, 0)
        pltpu.sync_copy(buf_v, outq_hbm.at[b, :, pl.ds(off + h * HALF, HALF)])


def _finalize_body(loss_ref, hist_ref, vq_ref, perp_ref):
    counts = jnp.sum(hist_ref[...], axis=0, keepdims=True)
    probs = counts * (1.0 / N_TOK)
    ent = -jnp.sum(probs * jnp.log(probs + 1e-10))
    perp_ref[...] = jnp.exp(ent).reshape(1, 1)
    mean_sq = loss_ref[...] * (1.0 / (N_TOK * DIM))
    vq_ref[...] = mean_sq + 0.25 * mean_sq


def kernel(inputs, weight):
    x3 = inputs.reshape(4, DIM, N_TOK // 4)

    idx3, loss_sum = pl.pallas_call(
        _argmin_body,
        grid=(N_TILES,),
        in_specs=[
            pl.BlockSpec((1, DIM, TILE), lambda i: (i // 16, 0, i % 16)),
            pl.BlockSpec((NUM_EMB, DIM), lambda i: (0, 0)),
        ],
        out_specs=[
            pl.BlockSpec((1, 1, TILE), lambda i: (i, 0, 0)),
            pl.BlockSpec((1, 1), lambda i: (0, 0)),
        ],
        out_shape=[
            jax.ShapeDtypeStruct((N_TILES, 1, TILE), jnp.int32),
            jax.ShapeDtypeStruct((1, 1), jnp.float32),
        ],
    )(x3, weight)
    indices = idx3.reshape(N_TOK)

    mesh = plsc.VectorSubcoreMesh(
        core_axis_name="c", subcore_axis_name="s", num_cores=2, num_subcores=16
    )
    outq, hist = pl.kernel(
        _sc_body,
        out_type=[
            jax.ShapeDtypeStruct((4, DIM, N_TOK // 4), jnp.float32),
            jax.ShapeDtypeStruct((NW, NUM_EMB), jnp.float32),
        ],
        mesh=mesh,
        compiler_params=pltpu.CompilerParams(needs_layout_passes=False),
        scratch_types=[
            pltpu.VMEM((TOK_PER_W,), jnp.int32),
            pltpu.VMEM((NUM_EMB * DIM,), jnp.float32),
            pltpu.VMEM((DIM, HALF), jnp.float32),
            pltpu.VMEM((NUM_EMB,), jnp.float32),
        ],
    )(indices, weight.reshape(-1))

    vq_arr, perp_arr = pl.pallas_call(
        _finalize_body,
        in_specs=[
            pl.BlockSpec((1, 1), lambda: (0, 0)),
            pl.BlockSpec((NW, NUM_EMB), lambda: (0, 0)),
        ],
        out_specs=[
            pl.BlockSpec((1, 1), lambda: (0, 0)),
            pl.BlockSpec((1, 1), lambda: (0, 0)),
        ],
        out_shape=[
            jax.ShapeDtypeStruct((1, 1), jnp.float32),
            jax.ShapeDtypeStruct((1, 1), jnp.float32),
        ],
    )(loss_sum, hist)

    quantized = outq.reshape(4, DIM, 8, 32, 32)
    return quantized, vq_arr[0, 0], perp_arr[0, 0], indices


# speedup vs baseline: 1.3164x; 1.3164x over previous
import functools

import jax
import jax.numpy as jnp
from jax import lax
from jax.experimental import pallas as pl
from jax.experimental.pallas import tpu as pltpu
from jax.experimental.pallas import tpu_sc as plsc

NUM_EMB = 1024
DIM = 64
N_TOK = 4 * 8 * 32 * 32
TILE = 1024
N_TILES = N_TOK // TILE
TILES_PER_B = (N_TOK // 4) // TILE
NW = 32
TOK_PER_W = N_TOK // NW
HALF = TOK_PER_W // 2


def _argmin_body(x_ref, w_ref, idx_ref, loss_ref):
    x = x_ref[0]
    xt = x.T
    w = w_ref[...]
    mm = lax.dot_general(
        xt, w, (((1,), (1,)), ((), ())),
        preferred_element_type=jnp.float32,
        precision=lax.Precision.DEFAULT,
    )
    xsq = jnp.sum(xt * xt, axis=1, keepdims=True)
    wsq = jnp.sum(w * w, axis=1)[None, :]
    d = (xsq + wsq) - 2.0 * mm
    mind = jnp.min(d, axis=1, keepdims=True)
    iota = lax.broadcasted_iota(jnp.int32, d.shape, 1)
    pick = jnp.where(d == mind, iota, jnp.int32(2**30))
    idx_ref[0, 0, :] = jnp.min(pick, axis=1)

    @pl.when(pl.program_id(0) == 0)
    def _():
        loss_ref[...] = jnp.zeros((1, 1), jnp.float32)

    loss_ref[...] += jnp.sum(mind).reshape(1, 1)


def _sc_body(idx_hbm, w_hbm, outq_hbm, hist_hbm, idx_v, w_v, buf_v, hist_v):
    cid = lax.axis_index("c")
    sid = lax.axis_index("s")
    wid = sid * 2 + cid
    pltpu.sync_copy(w_hbm, w_v)
    base = wid * TOK_PER_W
    pltpu.sync_copy(idx_hbm.at[pl.ds(base, TOK_PER_W)], idx_v)

    @plsc.parallel_loop(0, NUM_EMB // 16)
    def _zero(i):
        hist_v[pl.ds(i * 16, 16)] = jnp.zeros((16,), jnp.float32)

    ones = jnp.ones((16,), jnp.float32)

    @plsc.parallel_loop(0, TOK_PER_W // 16)
    def _hist(i):
        iv = idx_v[pl.ds(i * 16, 16)]
        plsc.addupdate_scatter(hist_v, [iv], ones)

    pltpu.sync_copy(hist_v, hist_hbm.at[wid])

    b = wid // 8
    off = (wid % 8) * TOK_PER_W
    for h in range(2):
        @plsc.parallel_loop(0, HALF // 16, unroll=2)
        def _gather(j):
            iv = idx_v[pl.ds(h * HALF + j * 16, 16)] * DIM
            for c in range(DIM):
                buf_v[c, pl.ds(j * 16, 16)] = plsc.load_gather(w_v, [iv + c])

        pltpu.sync_copy(buf_v, outq_hbm.at[b, :, pl.ds(off + h * HALF, HALF)])


def _finalize_body(loss_ref, hist_ref, vq_ref, perp_ref):
    counts = jnp.sum(hist_ref[...], axis=0, keepdims=True)
    probs = counts * (1.0 / N_TOK)
    ent = -jnp.sum(probs * jnp.log(probs + 1e-10))
    perp_ref[...] = jnp.exp(ent).reshape(1, 1)
    mean_sq = loss_ref[...] * (1.0 / (N_TOK * DIM))
    vq_ref[...] = mean_sq + 0.25 * mean_sq


def kernel(inputs, weight):
    x3 = inputs.reshape(4, DIM, N_TOK // 4)

    idx3, loss_sum = pl.pallas_call(
        _argmin_body,
        grid=(N_TILES,),
        in_specs=[
            pl.BlockSpec(
                (1, DIM, TILE),
                lambda i: (i // TILES_PER_B, 0, i % TILES_PER_B),
            ),
            pl.BlockSpec((NUM_EMB, DIM), lambda i: (0, 0)),
        ],
        out_specs=[
            pl.BlockSpec((1, 1, TILE), lambda i: (i, 0, 0)),
            pl.BlockSpec((1, 1), lambda i: (0, 0)),
        ],
        out_shape=[
            jax.ShapeDtypeStruct((N_TILES, 1, TILE), jnp.int32),
            jax.ShapeDtypeStruct((1, 1), jnp.float32),
        ],
    )(x3, weight)
    indices = idx3.reshape(N_TOK)

    mesh = plsc.VectorSubcoreMesh(
        core_axis_name="c", subcore_axis_name="s", num_cores=2, num_subcores=16
    )
    outq, hist = pl.kernel(
        _sc_body,
        out_type=[
            jax.ShapeDtypeStruct((4, DIM, N_TOK // 4), jnp.float32),
            jax.ShapeDtypeStruct((NW, NUM_EMB), jnp.float32),
        ],
        mesh=mesh,
        compiler_params=pltpu.CompilerParams(needs_layout_passes=False),
        scratch_types=[
            pltpu.VMEM((TOK_PER_W,), jnp.int32),
            pltpu.VMEM((NUM_EMB * DIM,), jnp.float32),
            pltpu.VMEM((DIM, HALF), jnp.float32),
            pltpu.VMEM((NUM_EMB,), jnp.float32),
        ],
    )(indices, weight.reshape(-1))

    vq_arr, perp_arr = pl.pallas_call(
        _finalize_body,
        in_specs=[
            pl.BlockSpec((1, 1), lambda: (0, 0)),
            pl.BlockSpec((NW, NUM_EMB), lambda: (0, 0)),
        ],
        out_specs=[
            pl.BlockSpec((1, 1), lambda: (0, 0)),
            pl.BlockSpec((1, 1), lambda: (0, 0)),
        ],
        out_shape=[
            jax.ShapeDtypeStruct((1, 1), jnp.float32),
            jax.ShapeDtypeStruct((1, 1), jnp.float32),
        ],
    )(loss_sum, hist)

    quantized = outq.reshape(4, DIM, 8, 32, 32)
    return quantized, vq_arr[0, 0], perp_arr[0, 0], indices


# speedup vs baseline: 1.8905x; 1.4361x over previous
import functools

import jax
import jax.numpy as jnp
from jax import lax
from jax.experimental import pallas as pl
from jax.experimental.pallas import tpu as pltpu
from jax.experimental.pallas import tpu_sc as plsc

NUM_EMB = 1024
DIM = 64
N_TOK = 4 * 8 * 32 * 32
TILE = 1024
N_TILES = N_TOK // TILE
TILES_PER_B = (N_TOK // 4) // TILE
NW = 32
TOK_PER_W = N_TOK // NW
HALF = TOK_PER_W // 2


CHUNK = 128
N_CHUNKS = NUM_EMB // CHUNK


SUB = 8
N_CH = NUM_EMB // SUB


def _argmin_body(x_ref, w_ref, idx_ref, loss_ref, w2_ref, wsqb_ref):
    i = pl.program_id(0)
    x = x_ref[0]
    w = w_ref[...]

    @pl.when(i == 0)
    def _():
        w2_ref[...] = -2.0 * w
        wsq = jnp.sum(w * w, axis=1, keepdims=True)
        wsqb_ref[...] = jnp.broadcast_to(wsq, (NUM_EMB, TILE))
        loss_ref[...] = jnp.zeros((1, 1), jnp.float32)

    mm2 = lax.dot_general(
        w2_ref[...], x, (((1,), (0,)), ((), ())),
        preferred_element_type=jnp.float32,
        precision=lax.Precision.DEFAULT,
    )

    xt = x.T
    xsq = jnp.sum(xt * xt, axis=1, keepdims=True)
    xsqb = jnp.broadcast_to(xsq.T, (SUB, TILE))

    m = None
    aj = None
    for j in range(N_CH):
        sl = slice(j * SUB, (j + 1) * SUB)
        dj = (wsqb_ref[sl, :] + xsqb) + mm2[sl, :]
        if j == 0:
            m = dj
            aj = jnp.zeros((SUB, TILE), jnp.float32)
        else:
            cmp = dj < m
            m = jnp.minimum(m, dj)
            aj = jnp.where(cmp, jnp.float32(j), aj)

    subiota = lax.broadcasted_iota(jnp.int32, (SUB, TILE), 0).astype(jnp.float32)
    code = aj * jnp.float32(SUB) + subiota
    mind8 = jnp.min(m, axis=0, keepdims=True)
    pick = jnp.where(m == jnp.broadcast_to(mind8, (SUB, TILE)), code,
                     jnp.float32(1e9))
    idx_ref[0, 0, :] = jnp.min(pick, axis=0).astype(jnp.int32)
    loss_ref[...] += jnp.sum(mind8).reshape(1, 1)


def _sc_body(idx_hbm, w_hbm, outq_hbm, hist_hbm, idx_v, w_v, buf_v, hist_v):
    cid = lax.axis_index("c")
    sid = lax.axis_index("s")
    wid = sid * 2 + cid
    pltpu.sync_copy(w_hbm, w_v)
    base = wid * TOK_PER_W
    pltpu.sync_copy(idx_hbm.at[pl.ds(base, TOK_PER_W)], idx_v)

    @plsc.parallel_loop(0, NUM_EMB // 16)
    def _zero(i):
        hist_v[pl.ds(i * 16, 16)] = jnp.zeros((16,), jnp.float32)

    ones = jnp.ones((16,), jnp.float32)

    @plsc.parallel_loop(0, TOK_PER_W // 16)
    def _hist(i):
        iv = idx_v[pl.ds(i * 16, 16)]
        plsc.addupdate_scatter(hist_v, [iv], ones)

    pltpu.sync_copy(hist_v, hist_hbm.at[wid])

    b = wid // 8
    off = (wid % 8) * TOK_PER_W
    for h in range(2):
        @plsc.parallel_loop(0, HALF // 16, unroll=2)
        def _gather(j):
            iv = idx_v[pl.ds(h * HALF + j * 16, 16)]
            for c in range(DIM):
                buf_v[c, pl.ds(j * 16, 16)] = plsc.load_gather(
                    w_v, [iv + c * NUM_EMB]
                )

        pltpu.sync_copy(buf_v, outq_hbm.at[b, :, pl.ds(off + h * HALF, HALF)])


def _finalize_body(loss_ref, hist_ref, vq_ref, perp_ref):
    counts = jnp.sum(hist_ref[...], axis=0, keepdims=True)
    probs = counts * (1.0 / N_TOK)
    ent = -jnp.sum(probs * jnp.log(probs + 1e-10))
    perp_ref[...] = jnp.exp(ent).reshape(1, 1)
    mean_sq = loss_ref[...] * (1.0 / (N_TOK * DIM))
    vq_ref[...] = mean_sq + 0.25 * mean_sq


def kernel(inputs, weight):
    x3 = inputs.reshape(4, DIM, N_TOK // 4)

    idx3_loss_wt = pl.pallas_call(
        _argmin_body,
        grid=(N_TILES,),
        in_specs=[
            pl.BlockSpec(
                (1, DIM, TILE),
                lambda i: (i // TILES_PER_B, 0, i % TILES_PER_B),
            ),
            pl.BlockSpec((NUM_EMB, DIM), lambda i: (0, 0)),
        ],
        out_specs=[
            pl.BlockSpec((1, 1, TILE), lambda i: (i, 0, 0)),
            pl.BlockSpec((1, 1), lambda i: (0, 0)),
        ],
        out_shape=[
            jax.ShapeDtypeStruct((N_TILES, 1, TILE), jnp.int32),
            jax.ShapeDtypeStruct((1, 1), jnp.float32),
        ],
        scratch_shapes=[
            pltpu.VMEM((NUM_EMB, DIM), jnp.float32),
            pltpu.VMEM((NUM_EMB, TILE), jnp.float32),
        ],
    )(x3, weight)
    idx3, loss_sum = idx3_loss_wt
    indices = idx3.reshape(N_TOK)
    wt = weight.T

    mesh = plsc.VectorSubcoreMesh(
        core_axis_name="c", subcore_axis_name="s", num_cores=2, num_subcores=16
    )
    outq, hist = pl.kernel(
        _sc_body,
        out_type=[
            jax.ShapeDtypeStruct((4, DIM, N_TOK // 4), jnp.float32),
            jax.ShapeDtypeStruct((NW, NUM_EMB), jnp.float32),
        ],
        mesh=mesh,
        compiler_params=pltpu.CompilerParams(needs_layout_passes=False),
        scratch_types=[
            pltpu.VMEM((TOK_PER_W,), jnp.int32),
            pltpu.VMEM((NUM_EMB * DIM,), jnp.float32),
            pltpu.VMEM((DIM, HALF), jnp.float32),
            pltpu.VMEM((NUM_EMB,), jnp.float32),
        ],
    )(indices, wt.reshape(-1))

    vq_arr, perp_arr = pl.pallas_call(
        _finalize_body,
        in_specs=[
            pl.BlockSpec((1, 1), lambda: (0, 0)),
            pl.BlockSpec((NW, NUM_EMB), lambda: (0, 0)),
        ],
        out_specs=[
            pl.BlockSpec((1, 1), lambda: (0, 0)),
            pl.BlockSpec((1, 1), lambda: (0, 0)),
        ],
        out_shape=[
            jax.ShapeDtypeStruct((1, 1), jnp.float32),
            jax.ShapeDtypeStruct((1, 1), jnp.float32),
        ],
    )(loss_sum, hist)

    quantized = outq.reshape(4, DIM, 8, 32, 32)
    return quantized, vq_arr[0, 0], perp_arr[0, 0], indices


# speedup vs baseline: 1.9708x; 1.0425x over previous
import functools

import jax
import jax.numpy as jnp
from jax import lax
from jax.experimental import pallas as pl
from jax.experimental.pallas import tpu as pltpu
from jax.experimental.pallas import tpu_sc as plsc

NUM_EMB = 1024
DIM = 64
N_TOK = 4 * 8 * 32 * 32
TILE = 4096
N_TILES = N_TOK // TILE
TILES_PER_B = (N_TOK // 4) // TILE
NW = 32
TOK_PER_W = N_TOK // NW
HALF = TOK_PER_W // 2


CHUNK = 128
N_CHUNKS = NUM_EMB // CHUNK


SUB = 8
N_CH = NUM_EMB // SUB


def _argmin_body(x_ref, w_ref, idx_ref, loss_ref, w2_ref, wsqb_ref):
    i = pl.program_id(0)
    x = x_ref[0]
    w = w_ref[...]

    @pl.when(i == 0)
    def _():
        w2_ref[...] = -2.0 * w
        wsq = jnp.sum(w * w, axis=1, keepdims=True)
        wsqb_ref[...] = jnp.broadcast_to(wsq, (NUM_EMB, TILE))
        loss_ref[...] = jnp.zeros((1, 1), jnp.float32)

    mm2 = lax.dot_general(
        w2_ref[...], x, (((1,), (0,)), ((), ())),
        preferred_element_type=jnp.float32,
        precision=lax.Precision.DEFAULT,
    )

    xt = x.T
    xsq = jnp.sum(xt * xt, axis=1, keepdims=True)
    xsqb = jnp.broadcast_to(xsq.T, (SUB, TILE))

    m = None
    aj = None
    for j in range(N_CH):
        sl = slice(j * SUB, (j + 1) * SUB)
        dj = (wsqb_ref[sl, :] + xsqb) + mm2[sl, :]
        if j == 0:
            m = dj
            aj = jnp.zeros((SUB, TILE), jnp.float32)
        else:
            cmp = dj < m
            m = jnp.minimum(m, dj)
            aj = jnp.where(cmp, jnp.float32(j), aj)

    subiota = lax.broadcasted_iota(jnp.int32, (SUB, TILE), 0).astype(jnp.float32)
    code = aj * jnp.float32(SUB) + subiota
    mind8 = jnp.min(m, axis=0, keepdims=True)
    pick = jnp.where(m == jnp.broadcast_to(mind8, (SUB, TILE)), code,
                     jnp.float32(1e9))
    idx_ref[0, 0, :] = jnp.min(pick, axis=0).astype(jnp.int32)
    loss_ref[...] += jnp.sum(mind8).reshape(1, 1)


def _sc_body(idx_hbm, w_hbm, outq_hbm, hist_hbm, idx_v, w_v, buf_v, hist_v):
    cid = lax.axis_index("c")
    sid = lax.axis_index("s")
    wid = sid * 2 + cid
    pltpu.sync_copy(w_hbm, w_v)
    base = wid * TOK_PER_W
    pltpu.sync_copy(idx_hbm.at[pl.ds(base, TOK_PER_W)], idx_v)

    @plsc.parallel_loop(0, NUM_EMB // 16)
    def _zero(i):
        hist_v[pl.ds(i * 16, 16)] = jnp.zeros((16,), jnp.float32)

    ones = jnp.ones((16,), jnp.float32)

    @plsc.parallel_loop(0, TOK_PER_W // 16)
    def _hist(i):
        iv = idx_v[pl.ds(i * 16, 16)]
        plsc.addupdate_scatter(hist_v, [iv], ones)

    pltpu.sync_copy(hist_v, hist_hbm.at[wid])

    b = wid // 8
    off = (wid % 8) * TOK_PER_W
    for h in range(2):
        @plsc.parallel_loop(0, HALF // 16, unroll=4)
        def _gather(j):
            iv = idx_v[pl.ds(h * HALF + j * 16, 16)]
            for c in range(DIM):
                buf_v[c, pl.ds(j * 16, 16)] = plsc.load_gather(
                    w_v, [iv + c * NUM_EMB]
                )

        pltpu.sync_copy(buf_v, outq_hbm.at[b, :, pl.ds(off + h * HALF, HALF)])


def _finalize_body(loss_ref, hist_ref, vq_ref, perp_ref):
    counts = jnp.sum(hist_ref[...], axis=0, keepdims=True)
    probs = counts * (1.0 / N_TOK)
    ent = -jnp.sum(probs * jnp.log(probs + 1e-10))
    perp_ref[...] = jnp.exp(ent).reshape(1, 1)
    mean_sq = loss_ref[...] * (1.0 / (N_TOK * DIM))
    vq_ref[...] = mean_sq + 0.25 * mean_sq


def kernel(inputs, weight):
    x3 = inputs.reshape(4, DIM, N_TOK // 4)

    idx3_loss_wt = pl.pallas_call(
        _argmin_body,
        grid=(N_TILES,),
        in_specs=[
            pl.BlockSpec(
                (1, DIM, TILE),
                lambda i: (i // TILES_PER_B, 0, i % TILES_PER_B),
            ),
            pl.BlockSpec((NUM_EMB, DIM), lambda i: (0, 0)),
        ],
        out_specs=[
            pl.BlockSpec((1, 1, TILE), lambda i: (i, 0, 0)),
            pl.BlockSpec((1, 1), lambda i: (0, 0)),
        ],
        out_shape=[
            jax.ShapeDtypeStruct((N_TILES, 1, TILE), jnp.int32),
            jax.ShapeDtypeStruct((1, 1), jnp.float32),
        ],
        scratch_shapes=[
            pltpu.VMEM((NUM_EMB, DIM), jnp.float32),
            pltpu.VMEM((NUM_EMB, TILE), jnp.float32),
        ],
    )(x3, weight)
    idx3, loss_sum = idx3_loss_wt
    indices = idx3.reshape(N_TOK)
    wt = weight.T

    mesh = plsc.VectorSubcoreMesh(
        core_axis_name="c", subcore_axis_name="s", num_cores=2, num_subcores=16
    )
    outq, hist = pl.kernel(
        _sc_body,
        out_type=[
            jax.ShapeDtypeStruct((4, DIM, N_TOK // 4), jnp.float32),
            jax.ShapeDtypeStruct((NW, NUM_EMB), jnp.float32),
        ],
        mesh=mesh,
        compiler_params=pltpu.CompilerParams(needs_layout_passes=False),
        scratch_types=[
            pltpu.VMEM((TOK_PER_W,), jnp.int32),
            pltpu.VMEM((NUM_EMB * DIM,), jnp.float32),
            pltpu.VMEM((DIM, HALF), jnp.float32),
            pltpu.VMEM((NUM_EMB,), jnp.float32),
        ],
    )(indices, wt.reshape(-1))

    vq_arr, perp_arr = pl.pallas_call(
        _finalize_body,
        in_specs=[
            pl.BlockSpec((1, 1), lambda: (0, 0)),
            pl.BlockSpec((NW, NUM_EMB), lambda: (0, 0)),
        ],
        out_specs=[
            pl.BlockSpec((1, 1), lambda: (0, 0)),
            pl.BlockSpec((1, 1), lambda: (0, 0)),
        ],
        out_shape=[
            jax.ShapeDtypeStruct((1, 1), jnp.float32),
            jax.ShapeDtypeStruct((1, 1), jnp.float32),
        ],
    )(loss_sum, hist)

    quantized = outq.reshape(4, DIM, 8, 32, 32)
    return quantized, vq_arr[0, 0], perp_arr[0, 0], indices


# speedup vs baseline: 1.9891x; 1.0093x over previous
import functools

import jax
import jax.numpy as jnp
from jax import lax
from jax.experimental import pallas as pl
from jax.experimental.pallas import tpu as pltpu
from jax.experimental.pallas import tpu_sc as plsc

NUM_EMB = 1024
DIM = 64
N_TOK = 4 * 8 * 32 * 32
TILE = 4096
N_TILES = N_TOK // TILE
TILES_PER_B = (N_TOK // 4) // TILE
NW = 32
TOK_PER_W = N_TOK // NW
HALF = TOK_PER_W // 2


CHUNK = 128
N_CHUNKS = NUM_EMB // CHUNK


SUB = 8
N_CH = NUM_EMB // SUB


def _argmin_body(x_ref, w_ref, idx_ref, loss_ref, w2_ref, wsqb_ref):
    i = pl.program_id(0)
    x = x_ref[0]
    w = w_ref[...]

    @pl.when(i == 0)
    def _():
        w2_ref[...] = -2.0 * w
        wsq = jnp.sum(w * w, axis=1, keepdims=True)
        wsqb_ref[...] = jnp.broadcast_to(wsq, (NUM_EMB, TILE))
        loss_ref[...] = jnp.zeros((1, 1), jnp.float32)

    mm2 = lax.dot_general(
        w2_ref[...], x, (((1,), (0,)), ((), ())),
        preferred_element_type=jnp.float32,
        precision=lax.Precision.DEFAULT,
    )

    xt = x.T
    xsq = jnp.sum(xt * xt, axis=1, keepdims=True)
    xsqb = jnp.broadcast_to(xsq.T, (SUB, TILE))

    m = None
    aj = None
    for j in range(N_CH):
        sl = slice(j * SUB, (j + 1) * SUB)
        dj = (wsqb_ref[sl, :] + xsqb) + mm2[sl, :]
        if j == 0:
            m = dj
            aj = jnp.zeros((SUB, TILE), jnp.float32)
        else:
            cmp = dj < m
            m = jnp.minimum(m, dj)
            aj = jnp.where(cmp, jnp.float32(j), aj)

    subiota = lax.broadcasted_iota(jnp.int32, (SUB, TILE), 0).astype(jnp.float32)
    code = aj * jnp.float32(SUB) + subiota
    mind8 = jnp.min(m, axis=0, keepdims=True)
    pick = jnp.where(m == jnp.broadcast_to(mind8, (SUB, TILE)), code,
                     jnp.float32(1e9))
    idx_ref[0, 0, :] = jnp.min(pick, axis=0).astype(jnp.int32)
    loss_ref[...] += jnp.sum(mind8).reshape(1, 1)


def _sc_body(idx_hbm, w_hbm, outq_hbm, hist_hbm, idx_v, w_v, buf_v, hist_v):
    cid = lax.axis_index("c")
    sid = lax.axis_index("s")
    wid = sid * 2 + cid
    pltpu.sync_copy(w_hbm, w_v)
    base = wid * TOK_PER_W
    pltpu.sync_copy(idx_hbm.at[pl.ds(base, TOK_PER_W)], idx_v)

    @plsc.parallel_loop(0, NUM_EMB // 16)
    def _zero(i):
        hist_v[pl.ds(i * 16, 16)] = jnp.zeros((16,), jnp.float32)

    ones = jnp.ones((16,), jnp.float32)

    @plsc.parallel_loop(0, TOK_PER_W // 16)
    def _hist(i):
        iv = idx_v[pl.ds(i * 16, 16)]
        plsc.addupdate_scatter(hist_v, [iv], ones)

    pltpu.sync_copy(hist_v, hist_hbm.at[wid])

    b = wid // 8
    off = (wid % 8) * TOK_PER_W
    for h in range(2):
        @plsc.parallel_loop(0, HALF // 16, unroll=2)
        def _gather(j):
            iv = idx_v[pl.ds(h * HALF + j * 16, 16)]
            for c in range(DIM):
                buf_v[c, pl.ds(j * 16, 16)] = plsc.load_gather(
                    w_v, [iv + c * NUM_EMB]
                )

        pltpu.sync_copy(buf_v, outq_hbm.at[b, :, pl.ds(off + h * HALF, HALF)])


def _finalize_body(loss_ref, hist_ref, vq_ref, perp_ref):
    counts = jnp.sum(hist_ref[...], axis=0, keepdims=True)
    probs = counts * (1.0 / N_TOK)
    ent = -jnp.sum(probs * jnp.log(probs + 1e-10))
    perp_ref[...] = jnp.exp(ent).reshape(1, 1)
    mean_sq = loss_ref[...] * (1.0 / (N_TOK * DIM))
    vq_ref[...] = mean_sq + 0.25 * mean_sq


def kernel(inputs, weight):
    x3 = inputs.reshape(4, DIM, N_TOK // 4)

    idx3_loss_wt = pl.pallas_call(
        _argmin_body,
        grid=(N_TILES,),
        in_specs=[
            pl.BlockSpec(
                (1, DIM, TILE),
                lambda i: (i // TILES_PER_B, 0, i % TILES_PER_B),
            ),
            pl.BlockSpec((NUM_EMB, DIM), lambda i: (0, 0)),
        ],
        out_specs=[
            pl.BlockSpec((1, 1, TILE), lambda i: (i, 0, 0)),
            pl.BlockSpec((1, 1), lambda i: (0, 0)),
        ],
        out_shape=[
            jax.ShapeDtypeStruct((N_TILES, 1, TILE), jnp.int32),
            jax.ShapeDtypeStruct((1, 1), jnp.float32),
        ],
        scratch_shapes=[
            pltpu.VMEM((NUM_EMB, DIM), jnp.float32),
            pltpu.VMEM((NUM_EMB, TILE), jnp.float32),
        ],
    )(x3, weight)
    idx3, loss_sum = idx3_loss_wt
    indices = idx3.reshape(N_TOK)
    wt = weight.T

    mesh = plsc.VectorSubcoreMesh(
        core_axis_name="c", subcore_axis_name="s", num_cores=2, num_subcores=16
    )
    outq, hist = pl.kernel(
        _sc_body,
        out_type=[
            jax.ShapeDtypeStruct((4, DIM, N_TOK // 4), jnp.float32),
            jax.ShapeDtypeStruct((NW, NUM_EMB), jnp.float32),
        ],
        mesh=mesh,
        compiler_params=pltpu.CompilerParams(needs_layout_passes=False),
        scratch_types=[
            pltpu.VMEM((TOK_PER_W,), jnp.int32),
            pltpu.VMEM((NUM_EMB * DIM,), jnp.float32),
            pltpu.VMEM((DIM, HALF), jnp.float32),
            pltpu.VMEM((NUM_EMB,), jnp.float32),
        ],
    )(indices, wt.reshape(-1))

    vq_arr, perp_arr = pl.pallas_call(
        _finalize_body,
        in_specs=[
            pl.BlockSpec((1, 1), lambda: (0, 0)),
            pl.BlockSpec((NW, NUM_EMB), lambda: (0, 0)),
        ],
        out_specs=[
            pl.BlockSpec((1, 1), lambda: (0, 0)),
            pl.BlockSpec((1, 1), lambda: (0, 0)),
        ],
        out_shape=[
            jax.ShapeDtypeStruct((1, 1), jnp.float32),
            jax.ShapeDtypeStruct((1, 1), jnp.float32),
        ],
    )(loss_sum, hist)

    quantized = outq.reshape(4, DIM, 8, 32, 32)
    return quantized, vq_arr[0, 0], perp_arr[0, 0], indices
